# Initial kernel scaffold; baseline (speedup 1.0000x reference)
import functools

import jax
import jax.numpy as jnp
from jax import lax
from jax.experimental import pallas as pl
from jax.experimental.pallas import tpu as pltpu
from jax.experimental.pallas import tpu_sc as plsc

NC = 2
NS = 16
NW = NC * NS
B = 128
LANES = 16


def _sc_segment_sum(nb, np_pad, h):
  rows_per_tile = np_pad // NS
  chunks_per_tile = rows_per_tile // B

  mesh = plsc.VectorSubcoreMesh(core_axis_name="c", subcore_axis_name="s",
                                num_cores=NC, num_subcores=NS)

  @functools.partial(
      pl.kernel,
      out_type=(
          jax.ShapeDtypeStruct((NC, np_pad, h), jnp.float32),
          jax.ShapeDtypeStruct((NC, np_pad, LANES), jnp.float32),
      ),
      mesh=mesh,
      scratch_types=[
          pltpu.VMEM((nb, B), jnp.int32),
          pltpu.VMEM((nb, B), jnp.int32),
          pltpu.VMEM((B, h), jnp.float32),
          pltpu.VMEM((B, LANES), jnp.float32),
          pltpu.VMEM((B, h), jnp.float32),
          pltpu.VMEM((B, LANES), jnp.float32),
          pltpu.VMEM_SHARED((np_pad, h), jnp.float32),
          pltpu.VMEM_SHARED((np_pad, LANES), jnp.float32),
          pltpu.SemaphoreType.DMA,
      ],
  )
  def k(xc_hbm, src_hbm, dst_hbm, acc_out, cnt_out,
        src_v, dst_v, rows_v, ones_v, z_h, z_n, acc_sh, cnt_sh, sem):
    c = lax.axis_index("c")
    s = lax.axis_index("s")
    wid = s * NC + c
    base = s * rows_per_tile

    def init_row(i, _):
      for q in range(h // LANES):
        z_h[i, pl.ds(q * LANES, LANES)] = jnp.zeros((LANES,), jnp.float32)
      z_n[i] = jnp.zeros((LANES,), jnp.float32)
      ones_v[i] = jnp.full((LANES,), 1.0, jnp.float32)
      return _
    lax.fori_loop(0, B, init_row, None)

    for kk in range(chunks_per_tile):
      pltpu.sync_copy(z_h, acc_sh.at[pl.ds(base + kk * B, B)])
      pltpu.sync_copy(z_n, cnt_sh.at[pl.ds(base + kk * B, B)])
    plsc.subcore_barrier()

    pltpu.sync_copy(src_hbm.at[wid], src_v)
    pltpu.sync_copy(dst_hbm.at[wid], dst_v)

    def body(j, _):
      pltpu.async_copy(xc_hbm.at[dst_v.at[j]], rows_v, sem).wait()
      pltpu.sync_copy(rows_v, acc_sh.at[src_v.at[j]], add=True)
      pltpu.sync_copy(ones_v, cnt_sh.at[src_v.at[j]], add=True)
      return _
    lax.fori_loop(0, nb, body, None)
    plsc.subcore_barrier()

    for kk in range(chunks_per_tile):
      r0 = base + kk * B
      pltpu.sync_copy(acc_sh.at[pl.ds(r0, B)], acc_out.at[c, pl.ds(r0, B)])
    pltpu.sync_copy(cnt_sh.at[pl.ds(base, rows_per_tile)],
                    cnt_out.at[c, pl.ds(base, rows_per_tile)])

  return k


def _dense_body(xp_ref, acc_ref, cnt_ref, w1_ref, b1_ref, w2_ref, b2_ref,
                wu_ref, bu_ref, g_ref, bt_ref, out_ref):
  xp = xp_ref[...]
  ssum = acc_ref[0] + acc_ref[1]
  cnt = cnt_ref[0][:, :1] + cnt_ref[1][:, :1]
  hid = jnp.maximum(
      jnp.dot(xp, w1_ref[...], preferred_element_type=jnp.float32) + b1_ref[...],
      0.0)
  pred = jnp.dot(hid, w2_ref[...], preferred_element_type=jnp.float32) + b2_ref[...]
  agg = (ssum - cnt * pred) / jnp.maximum(cnt, 1.0)
  upd = jnp.dot(agg, wu_ref[...], preferred_element_type=jnp.float32) + bu_ref[...]
  t = xp + upd
  m = jnp.mean(t, axis=1, keepdims=True)
  v = jnp.mean((t - m) * (t - m), axis=1, keepdims=True)
  out_ref[...] = (t - m) * lax.rsqrt(v + 1e-5) * g_ref[...] + bt_ref[...]


def kernel(x_parent, x_child, edge_index, W1, b1, W2, b2, Wu, bu, gamma, beta):
  np_, h = x_parent.shape
  e = edge_index.shape[1]

  np_pad = -(-(np_ + 1) // (NS * B)) * (NS * B)
  chunk = NW * B
  e_pad = -(-e // chunk) * chunk
  nb = e_pad // chunk

  src = edge_index[0]
  dst = edge_index[1]
  pad = e_pad - e
  if pad:
    src = jnp.concatenate([src, jnp.full((pad,), np_, jnp.int32)])
    dst = jnp.concatenate([dst, jnp.zeros((pad,), jnp.int32)])
  src3 = src.reshape(NW, nb, B)
  dst3 = dst.reshape(NW, nb, B)

  acc, cnt = _sc_segment_sum(nb, np_pad, h)(x_child, src3, dst3)

  r = 1000 if np_ % 1000 == 0 else np_
  grid = (np_ // r,)
  new_parent = pl.pallas_call(
      _dense_body,
      grid=grid,
      in_specs=[
          pl.BlockSpec((r, h), lambda i: (i, 0)),
          pl.BlockSpec((NC, r, h), lambda i: (0, i, 0)),
          pl.BlockSpec((NC, r, LANES), lambda i: (0, i, 0)),
          pl.BlockSpec((h, h), lambda i: (0, 0)),
          pl.BlockSpec((1, h), lambda i: (0, 0)),
          pl.BlockSpec((h, h), lambda i: (0, 0)),
          pl.BlockSpec((1, h), lambda i: (0, 0)),
          pl.BlockSpec((h, h), lambda i: (0, 0)),
          pl.BlockSpec((1, h), lambda i: (0, 0)),
          pl.BlockSpec((1, h), lambda i: (0, 0)),
          pl.BlockSpec((1, h), lambda i: (0, 0)),
      ],
      out_specs=pl.BlockSpec((r, h), lambda i: (i, 0)),
      out_shape=jax.ShapeDtypeStruct((np_, h), jnp.float32),
  )(x_parent, acc[:, :np_], cnt[:, :np_],
    W1, b1.reshape(1, h), W2, b2.reshape(1, h), Wu, bu.reshape(1, h),
    gamma.reshape(1, h), beta.reshape(1, h))

  return (new_parent, x_child)


# speedup vs baseline: 3.9293x; 3.9293x over previous
import functools

import jax
import jax.numpy as jnp
from jax import lax
from jax.experimental import pallas as pl
from jax.experimental.pallas import tpu as pltpu
from jax.experimental.pallas import tpu_sc as plsc

NC = 2
NS = 16
NW = NC * NS
B = 64
LANES = 16
CW = 128


def _sc_segment_sum(nb, np_pad, h):
  n_chunks = np_pad // B
  np_cpad = np_pad // CW
  cnt_rpt = 8
  cnt_nchunks = np_cpad // cnt_rpt

  mesh = plsc.VectorSubcoreMesh(core_axis_name="c", subcore_axis_name="s",
                                num_cores=NC, num_subcores=NS)

  @functools.partial(
      pl.kernel,
      out_type=(
          jax.ShapeDtypeStruct((NC, np_pad, h), jnp.float32),
          jax.ShapeDtypeStruct((NC, np_cpad, CW), jnp.float32),
      ),
      mesh=mesh,
      scratch_types=[
          pltpu.VMEM((B,), jnp.int32),
          pltpu.VMEM((B,), jnp.int32),
          pltpu.VMEM((B,), jnp.int32),
          pltpu.VMEM((B,), jnp.int32),
          pltpu.VMEM((B, h), jnp.float32),
          pltpu.VMEM((B, CW), jnp.float32),
          pltpu.VMEM_SHARED((np_pad, h), jnp.float32),
          pltpu.VMEM_SHARED((np_cpad, CW), jnp.float32),
          pltpu.SemaphoreType.DMA,
      ],
  )
  def k(xc_hbm, oh_hbm, src_hbm, dst_hbm, acc_out, cnt_out,
        src_v, dst_v, hi_v, lo_v, rows_v, pay_v, acc_sh, cnt_sh, sem):
    c = lax.axis_index("c")
    s = lax.axis_index("s")
    wid = s * NC + c

    def init_row(i, _):
      for q in range(h // LANES):
        rows_v[i, pl.ds(q * LANES, LANES)] = jnp.zeros((LANES,), jnp.float32)
      return _
    lax.fori_loop(0, B, init_row, None)

    @pl.loop(s, n_chunks, step=NS)
    def zero_chunk(kk):
      pltpu.sync_copy(rows_v, acc_sh.at[pl.ds(kk * B, B)])

    @pl.loop(s, cnt_nchunks, step=NS)
    def zero_cnt(kk):
      pltpu.sync_copy(rows_v.at[pl.ds(0, cnt_rpt)], cnt_sh.at[pl.ds(kk * cnt_rpt, cnt_rpt)])
    plsc.subcore_barrier()

    def body(j, _):
      pltpu.sync_copy(src_hbm.at[wid, j], src_v)
      pltpu.sync_copy(dst_hbm.at[wid, j], dst_v)
      for q in range(B // LANES):
        s16 = src_v[pl.ds(q * LANES, LANES)]
        hi_v[pl.ds(q * LANES, LANES)] = lax.shift_right_logical(s16, 7)
        lo_v[pl.ds(q * LANES, LANES)] = lax.bitwise_and(s16, CW - 1)
      pltpu.async_copy(xc_hbm.at[dst_v], rows_v, sem).wait()
      pltpu.sync_copy(rows_v, acc_sh.at[src_v], add=True)
      pltpu.async_copy(oh_hbm.at[lo_v], pay_v, sem).wait()
      pltpu.sync_copy(pay_v, cnt_sh.at[hi_v], add=True)
      return _
    lax.fori_loop(0, nb, body, None)
    plsc.subcore_barrier()

    @pl.loop(s, n_chunks, step=NS)
    def out_chunk(kk):
      pltpu.sync_copy(acc_sh.at[pl.ds(kk * B, B)], rows_v)
      pltpu.sync_copy(rows_v, acc_out.at[c, pl.ds(kk * B, B)])

    @pl.loop(s, cnt_nchunks, step=NS)
    def out_cnt(kk):
      pltpu.sync_copy(cnt_sh.at[pl.ds(kk * cnt_rpt, cnt_rpt)], pay_v.at[pl.ds(0, cnt_rpt)])
      pltpu.sync_copy(pay_v.at[pl.ds(0, cnt_rpt)], cnt_out.at[c, pl.ds(kk * cnt_rpt, cnt_rpt)])

  return k


def _dense_body(xp_ref, acc_ref, cnt_ref, w1_ref, b1_ref, w2_ref, b2_ref,
                wu_ref, bu_ref, g_ref, bt_ref, out_ref):
  xp = xp_ref[...]
  ssum = acc_ref[0] + acc_ref[1]
  cnt = cnt_ref[0] + cnt_ref[1]
  hid = jnp.maximum(
      jnp.dot(xp, w1_ref[...], preferred_element_type=jnp.float32) + b1_ref[...],
      0.0)
  pred = jnp.dot(hid, w2_ref[...], preferred_element_type=jnp.float32) + b2_ref[...]
  agg = (ssum - cnt * pred) / jnp.maximum(cnt, 1.0)
  upd = jnp.dot(agg, wu_ref[...], preferred_element_type=jnp.float32) + bu_ref[...]
  t = xp + upd
  m = jnp.mean(t, axis=1, keepdims=True)
  v = jnp.mean((t - m) * (t - m), axis=1, keepdims=True)
  out_ref[...] = (t - m) * lax.rsqrt(v + 1e-5) * g_ref[...] + bt_ref[...]


def kernel(x_parent, x_child, edge_index, W1, b1, W2, b2, Wu, bu, gamma, beta):
  np_, h = x_parent.shape
  e = edge_index.shape[1]

  np_pad = -(-(np_ + 1) // (NS * B * 2)) * (NS * B * 2)
  chunk = NW * B
  e_pad = -(-e // chunk) * chunk
  nb = e_pad // chunk

  onehot = jnp.eye(CW, dtype=jnp.float32)

  src = edge_index[0]
  dst = edge_index[1]
  pad = e_pad - e
  if pad:
    src = jnp.concatenate([src, jnp.full((pad,), np_, jnp.int32)])
    dst = jnp.concatenate([dst, jnp.zeros((pad,), jnp.int32)])
  src3 = src.reshape(NW, nb, B)
  dst3 = dst.reshape(NW, nb, B)

  acc, cnt = _sc_segment_sum(nb, np_pad, h)(x_child, onehot, src3, dst3)
  cnt_col = cnt.reshape(NC, np_pad, 1)

  r = 1000 if np_ % 1000 == 0 else np_
  grid = (np_ // r,)
  new_parent = pl.pallas_call(
      _dense_body,
      grid=grid,
      in_specs=[
          pl.BlockSpec((r, h), lambda i: (i, 0)),
          pl.BlockSpec((NC, r, h), lambda i: (0, i, 0)),
          pl.BlockSpec((NC, r, 1), lambda i: (0, i, 0)),
          pl.BlockSpec((h, h), lambda i: (0, 0)),
          pl.BlockSpec((1, h), lambda i: (0, 0)),
          pl.BlockSpec((h, h), lambda i: (0, 0)),
          pl.BlockSpec((1, h), lambda i: (0, 0)),
          pl.BlockSpec((h, h), lambda i: (0, 0)),
          pl.BlockSpec((1, h), lambda i: (0, 0)),
          pl.BlockSpec((1, h), lambda i: (0, 0)),
          pl.BlockSpec((1, h), lambda i: (0, 0)),
      ],
      out_specs=pl.BlockSpec((r, h), lambda i: (i, 0)),
      out_shape=jax.ShapeDtypeStruct((np_, h), jnp.float32),
  )(x_parent, acc[:, :np_], cnt_col[:, :np_],
    W1, b1.reshape(1, h), W2, b2.reshape(1, h), Wu, bu.reshape(1, h),
    gamma.reshape(1, h), beta.reshape(1, h))

  return (new_parent, x_child)
